# baseline (speedup 1.0000x reference)
import functools

import jax
import jax.numpy as jnp
from jax import lax
from jax.experimental import pallas as pl
from jax.experimental.pallas import tpu as pltpu
from jax.experimental.pallas import tpu_sc as plsc

NTOK = 16384
DIM = 4096
NE = 64
BT = 1024
NSTEPS = NTOK // BT

NSLICE = 4
SLT = NTOK // NSLICE
NSTEPS_S = SLT // BT

NC = 2
NS = 16
NW = NC * NS
TPW = SLT // NW
CH = 128
NCH = TPW // CH

INT_MIN = -2147483648


def _scores_kernel(x_ref, wt_ref, b_ref, scores_ref, ssum_ref):
    logits = jnp.dot(x_ref[...], wt_ref[...], preferred_element_type=jnp.float32)
    logits = logits + b_ref[...]
    m = jnp.max(logits, axis=1, keepdims=True)
    e = jnp.exp(logits - m)
    scores = e / jnp.sum(e, axis=1, keepdims=True)
    scores_ref[...] = scores
    ssum_ref[0, 0:1, :] = jnp.sum(scores, axis=0, keepdims=True)


def _sc_topk_body(scores_hbm, out4_hbm, out1_hbm, msum_hbm,
                  sbuf, o4buf, o1buf, msbuf):
    wid = lax.axis_index("s") * NC + lax.axis_index("c")
    base = wid * TPW
    lane = lax.iota(jnp.int32, 16)

    msums = [jnp.zeros((16,), jnp.float32) for _ in range(4)]
    for c in range(NCH):
        row0 = base + c * CH
        pltpu.sync_copy(scores_hbm.at[pl.ds(row0, CH), :], sbuf)

        def body(t, carry):
            ms = list(carry)
            svs = [sbuf[t, pl.ds(16 * v, 16)] for v in range(4)]
            idxs = [lane + jnp.int32(16 * v) for v in range(4)]
            kvs = list(svs)
            masks = [None] * 4
            for r in range(4):
                mx = jnp.max(jnp.maximum(jnp.maximum(kvs[0], kvs[1]),
                                         jnp.maximum(kvs[2], kvs[3])))
                cands = [jnp.where(kvs[v] == mx, idxs[v], jnp.int32(NE))
                         for v in range(4)]
                midx = jnp.min(jnp.minimum(jnp.minimum(cands[0], cands[1]),
                                           jnp.minimum(cands[2], cands[3])))
                sels = [idxs[v] == midx for v in range(4)]
                if r == 0:
                    for v in range(4):
                        o1buf[t, pl.ds(16 * v, 16)] = jnp.where(
                            sels[v], svs[v], 0.0)
                        masks[v] = sels[v]
                else:
                    for v in range(4):
                        masks[v] = jnp.logical_or(masks[v], sels[v])
                kvs = [jnp.where(sels[v], -jnp.inf, kvs[v]) for v in range(4)]
            for v in range(4):
                o4buf[t, pl.ds(16 * v, 16)] = jnp.where(masks[v], svs[v], 0.0)
                ms[v] = ms[v] + jnp.where(masks[v], 1.0, 0.0)
            return tuple(ms)

        msums = list(lax.fori_loop(0, CH, body, tuple(msums)))
        pltpu.sync_copy(o4buf, out4_hbm.at[pl.ds(row0, CH), :])
        pltpu.sync_copy(o1buf, out1_hbm.at[pl.ds(row0, CH), :])

    for v in range(4):
        msbuf[pl.ds(16 * v, 16)] = msums[v]
    pltpu.sync_copy(msbuf, msum_hbm.at[wid])


_sc_topk = functools.partial(
    pl.kernel,
    out_type=[
        jax.ShapeDtypeStruct((SLT, NE), jnp.float32),
        jax.ShapeDtypeStruct((SLT, NE), jnp.float32),
        jax.ShapeDtypeStruct((NW, NE), jnp.float32),
    ],
    mesh=plsc.VectorSubcoreMesh(core_axis_name="c", subcore_axis_name="s"),
    compiler_params=pltpu.CompilerParams(needs_layout_passes=False),
    scratch_types=[
        pltpu.VMEM((CH, NE), jnp.float32),
        pltpu.VMEM((CH, NE), jnp.float32),
        pltpu.VMEM((CH, NE), jnp.float32),
        pltpu.VMEM((NE,), jnp.float32),
    ],
)(_sc_topk_body)


def _loss_kernel(ssum_ref, msum_ref, loss_ref):
    ssum = jnp.sum(ssum_ref[:, 0, :], axis=0, keepdims=True)
    msum = jnp.sum(msum_ref[...], axis=0, keepdims=True)
    n = jnp.float32(NTOK)
    loss_ref[...] = NE * jnp.sum(ssum * msum, axis=1, keepdims=True) / (n * n)


def _tc_scores(x_slice, wt, b2):
    return pl.pallas_call(
        _scores_kernel,
        grid=(NSTEPS_S,),
        in_specs=[
            pl.BlockSpec((BT, DIM), lambda i: (i, 0)),
            pl.BlockSpec((DIM, NE), lambda i: (0, 0)),
            pl.BlockSpec((1, NE), lambda i: (0, 0)),
        ],
        out_specs=[
            pl.BlockSpec((BT, NE), lambda i: (i, 0)),
            pl.BlockSpec((1, 1, NE), lambda i: (i, 0, 0)),
        ],
        out_shape=[
            jax.ShapeDtypeStruct((SLT, NE), jnp.float32),
            jax.ShapeDtypeStruct((NSTEPS_S, 1, NE), jnp.float32),
        ],
        compiler_params=pltpu.CompilerParams(
            dimension_semantics=("parallel",),
        ),
    )(x_slice, wt, b2)


@jax.jit
def _gate(x, wt, b2):
    out4s, out1s, msums, ssums = [], [], [], []
    for s in range(NSLICE):
        scores_s, ssum_s = _tc_scores(
            jax.lax.slice_in_dim(x, s * SLT, (s + 1) * SLT, axis=0), wt, b2)
        out4_s, out1_s, msum_s = _sc_topk(scores_s)
        out4s.append(out4_s)
        out1s.append(out1_s)
        msums.append(msum_s)
        ssums.append(ssum_s)

    out4 = jnp.concatenate(out4s, axis=0)
    out1 = jnp.concatenate(out1s, axis=0)
    ssum = jnp.concatenate(ssums, axis=0)
    msum = jnp.concatenate(msums, axis=0)

    loss = pl.pallas_call(
        _loss_kernel,
        out_shape=jax.ShapeDtypeStruct((1, 1), jnp.float32),
    )(ssum, msum)
    return out4, loss.reshape(()), out1


def kernel(x, W, b):
    return _gate(x, W.T, b.reshape(1, NE))


# speedup vs baseline: 3.1743x; 3.1743x over previous
import functools

import jax
import jax.numpy as jnp
from jax.experimental import pallas as pl
from jax.experimental.pallas import tpu as pltpu

NTOK = 16384
DIM = 4096
NE = 64
BT = 1024
NSTEPS = NTOK // BT


def _gate_kernel(x_ref, wt_ref, b_ref, out4_ref, out1_ref, sums_ref):
    logits = jnp.dot(x_ref[...], wt_ref[...], preferred_element_type=jnp.float32)
    logits = logits + b_ref[...]

    m = jnp.max(logits, axis=1, keepdims=True)
    e = jnp.exp(logits - m)
    scores = e / jnp.sum(e, axis=1, keepdims=True)

    iota = jax.lax.broadcasted_iota(jnp.int32, scores.shape, 1)
    key = (scores.view(jnp.int32) & jnp.int32(~0x3F)) | (jnp.int32(NE - 1) - iota)
    mask = None
    for k in range(4):
        mx = jnp.max(key, axis=1, keepdims=True)
        sel = key == mx
        if k == 0:
            out1_ref[...] = jnp.where(sel, scores, 0.0)
            mask = sel
        else:
            mask = jnp.logical_or(mask, sel)
        key = jnp.where(sel, jnp.int32(-2147483648), key)

    out4_ref[...] = jnp.where(mask, scores, 0.0)

    sums_ref[0, 0:1, :] = jnp.sum(scores, axis=0, keepdims=True)
    sums_ref[0, 1:2, :] = jnp.sum(mask.astype(jnp.float32), axis=0, keepdims=True)


def _loss_kernel(sums_ref, loss_ref):
    ssum = jnp.sum(sums_ref[:, 0, :], axis=0, keepdims=True)
    msum = jnp.sum(sums_ref[:, 1, :], axis=0, keepdims=True)
    n = jnp.float32(NTOK)
    loss_ref[...] = NE * jnp.sum(ssum * msum, axis=1, keepdims=True) / (n * n)


@jax.jit
def _gate(x, wt, b2):
    out4, out1, sums = pl.pallas_call(
        _gate_kernel,
        grid=(NSTEPS,),
        in_specs=[
            pl.BlockSpec((BT, DIM), lambda i: (i, 0)),
            pl.BlockSpec((DIM, NE), lambda i: (0, 0)),
            pl.BlockSpec((1, NE), lambda i: (0, 0)),
        ],
        out_specs=[
            pl.BlockSpec((BT, NE), lambda i: (i, 0)),
            pl.BlockSpec((BT, NE), lambda i: (i, 0)),
            pl.BlockSpec((1, 2, NE), lambda i: (i, 0, 0)),
        ],
        out_shape=[
            jax.ShapeDtypeStruct((NTOK, NE), jnp.float32),
            jax.ShapeDtypeStruct((NTOK, NE), jnp.float32),
            jax.ShapeDtypeStruct((NSTEPS, 2, NE), jnp.float32),
        ],
        compiler_params=pltpu.CompilerParams(
            dimension_semantics=("parallel",),
        ),
    )(x, wt, b2)
    loss = pl.pallas_call(
        _loss_kernel,
        out_shape=jax.ShapeDtypeStruct((1, 1), jnp.float32),
    )(sums)
    return out4, loss.reshape(()), out1


def kernel(x, W, b):
    return _gate(x, W.T, b.reshape(1, NE))
